# baseline (speedup 1.0000x reference)
import functools

import jax
import jax.numpy as jnp
from jax import lax
from jax.experimental import pallas as pl
from jax.experimental.pallas import tpu as pltpu
from jax.experimental.pallas import tpu_sc as plsc

_N = 8192
_V = 8192
_D = 64
_BM = 1024
_BN = 1024

_SC_NC = 2
_SC_NS = 16
_SC_NW = _SC_NC * _SC_NS
_B_PER_W = _N // _SC_NW
_D_PAD = 128


def _l2norm(t):
    n = jnp.linalg.norm(t, axis=-1, keepdims=True)
    n = jnp.maximum(n, 1e-12)
    return t / n


def _argmin_body(zb_ref, w_ref, idx_ref, dmin_ref, dsum_ref):
    zb = zb_ref[...]
    cur_min = jnp.full((_BM, 1), jnp.inf, jnp.float32)
    cur_idx = jnp.zeros((_BM, 1), jnp.int32)
    for j in range(_V // _BN):
        wb = w_ref[pl.ds(j * _BN, _BN), :]
        s = lax.dot_general(zb, wb, (((1,), (1,)), ((), ())),
                            preferred_element_type=jnp.float32)
        d = 2.0 - 2.0 * s
        m = jnp.min(d, axis=1, keepdims=True)
        col = lax.broadcasted_iota(jnp.int32, (_BM, _BN), 1)
        a = jnp.min(jnp.where(d == m, col, _BN), axis=1, keepdims=True)
        take = m < cur_min
        cur_idx = jnp.where(take, a + j * _BN, cur_idx)
        cur_min = jnp.where(take, m, cur_min)
    idx_ref[...] = cur_idx
    dmin_ref[...] = cur_min

    @pl.when(pl.program_id(0) == 0)
    def _():
        dsum_ref[0] = 0.0

    dsum_ref[0] += jnp.sum(cur_min)


def _distance_argmin(z_n, w_n):
    return pl.pallas_call(
        _argmin_body,
        grid=(_N // _BM,),
        in_specs=[
            pl.BlockSpec((_BM, _D), lambda i: (i, 0)),
            pl.BlockSpec((_V, _D), lambda i: (0, 0)),
        ],
        out_specs=[
            pl.BlockSpec((_BM, 1), lambda i: (i, 0)),
            pl.BlockSpec((_BM, 1), lambda i: (i, 0)),
            pl.BlockSpec(memory_space=pltpu.SMEM),
        ],
        out_shape=[
            jax.ShapeDtypeStruct((_N, 1), jnp.int32),
            jax.ShapeDtypeStruct((_N, 1), jnp.float32),
            jax.ShapeDtypeStruct((1,), jnp.float32),
        ],
    )(z_n, w_n)


@functools.lru_cache(maxsize=1)
def _make_gather_sc():
    @functools.partial(
        pl.kernel,
        mesh=plsc.VectorSubcoreMesh(core_axis_name="c", subcore_axis_name="s"),
        out_type=jax.ShapeDtypeStruct((_N, _D_PAD), jnp.float32),
        scratch_types=[
            pltpu.VMEM((_B_PER_W,), jnp.int32),
            pltpu.VMEM((_B_PER_W, _D_PAD), jnp.float32),
            pltpu.SemaphoreType.DMA,
        ],
    )
    def _gather_sc(table_hbm, idx_hbm, out_hbm, idx_v, rows_v, sem):
        wid = lax.axis_index("s") * _SC_NC + lax.axis_index("c")
        base = wid * _B_PER_W
        pltpu.sync_copy(idx_hbm.at[pl.ds(base, _B_PER_W)], idx_v)
        pltpu.async_copy(table_hbm.at[idx_v], rows_v, sem).wait()
        pltpu.sync_copy(rows_v, out_hbm.at[pl.ds(base, _B_PER_W)])

    return _gather_sc


def kernel(z, W):
    beta = 0.25
    z_n = _l2norm(z).reshape(-1, _D)
    w_n = _l2norm(W)
    idx, _dmin, dsum = _distance_argmin(z_n, w_n)
    idx_flat = idx.reshape(_N)
    w_n_pad = jnp.pad(w_n, ((0, 0), (0, _D_PAD - _D)))
    z_q = _make_gather_sc()(w_n_pad, idx_flat)
    z_q_out = z_q[:, :_D].reshape(z.shape)
    loss = (1.0 + beta) * dsum[0] / (_N * _D)
    encoding_indices = idx_flat.reshape(z.shape[:-1])
    return (z_q_out, loss, encoding_indices)


# speedup vs baseline: 1.0151x; 1.0151x over previous
import functools

import jax
import jax.numpy as jnp
from jax import lax
from jax.experimental import pallas as pl
from jax.experimental.pallas import tpu as pltpu
from jax.experimental.pallas import tpu_sc as plsc

_N = 8192
_V = 8192
_D = 64
_BM = 1024
_BN = 1024

_SC_NC = 2
_SC_NS = 16
_SC_NW = _SC_NC * _SC_NS
_B_PER_W = _N // _SC_NW
_D_PAD = 128


def _l2norm(t):
    n = jnp.linalg.norm(t, axis=-1, keepdims=True)
    n = jnp.maximum(n, 1e-12)
    return t / n


def _argmin_body(zb_ref, w_ref, idx_ref, dsum_ref, s_ref):
    zb = zb_ref[...]
    m = jnp.full((_BM, 1), -jnp.inf, jnp.float32)
    for j in range(_V // _BN):
        wb = w_ref[pl.ds(j * _BN, _BN), :]
        s = lax.dot_general(zb, wb, (((1,), (1,)), ((), ())),
                            preferred_element_type=jnp.float32)
        s_ref[:, pl.ds(j * _BN, _BN)] = s
        m = jnp.maximum(m, jnp.max(s, axis=1, keepdims=True))
    d_min = 2.0 - 2.0 * m
    colf = lax.broadcasted_iota(jnp.int32, (_BM, _BN), 1).astype(jnp.float32)
    run = jnp.full((_BM, 1), 3.0e4, jnp.float32)
    for j in range(_V // _BN):
        d = 2.0 - 2.0 * s_ref[:, pl.ds(j * _BN, _BN)]
        cand = jnp.where(d == d_min, colf, 2.0e4)
        cmin = jnp.min(cand, axis=1, keepdims=True) + float(j * _BN)
        run = jnp.minimum(run, cmin)
    idx_ref[...] = run.astype(jnp.int32)

    @pl.when(pl.program_id(0) == 0)
    def _():
        dsum_ref[0] = 0.0

    dsum_ref[0] += jnp.sum(d_min)


def _distance_argmin(z_n, w_n):
    return pl.pallas_call(
        _argmin_body,
        grid=(_N // _BM,),
        in_specs=[
            pl.BlockSpec((_BM, _D), lambda i: (i, 0)),
            pl.BlockSpec((_V, _D), lambda i: (0, 0)),
        ],
        out_specs=[
            pl.BlockSpec((_BM, 1), lambda i: (i, 0)),
            pl.BlockSpec(memory_space=pltpu.SMEM),
        ],
        out_shape=[
            jax.ShapeDtypeStruct((_N, 1), jnp.int32),
            jax.ShapeDtypeStruct((1,), jnp.float32),
        ],
        scratch_shapes=[pltpu.VMEM((_BM, _V), jnp.float32)],
    )(z_n, w_n)


@functools.lru_cache(maxsize=1)
def _make_gather_sc():
    @functools.partial(
        pl.kernel,
        mesh=plsc.VectorSubcoreMesh(core_axis_name="c", subcore_axis_name="s"),
        out_type=jax.ShapeDtypeStruct((_N, _D_PAD), jnp.float32),
        scratch_types=[
            pltpu.VMEM((_B_PER_W,), jnp.int32),
            pltpu.VMEM((_B_PER_W, _D_PAD), jnp.float32),
            pltpu.SemaphoreType.DMA,
        ],
    )
    def _gather_sc(table_hbm, idx_hbm, out_hbm, idx_v, rows_v, sem):
        wid = lax.axis_index("s") * _SC_NC + lax.axis_index("c")
        base = wid * _B_PER_W
        pltpu.sync_copy(idx_hbm.at[pl.ds(base, _B_PER_W)], idx_v)
        pltpu.async_copy(table_hbm.at[idx_v], rows_v, sem).wait()
        pltpu.sync_copy(rows_v, out_hbm.at[pl.ds(base, _B_PER_W)])

    return _gather_sc


def kernel(z, W):
    beta = 0.25
    z_n = _l2norm(z).reshape(-1, _D)
    w_n = _l2norm(W)
    idx, dsum = _distance_argmin(z_n, w_n)
    idx_flat = idx.reshape(_N)
    w_n_pad = jnp.pad(w_n, ((0, 0), (0, _D_PAD - _D)))
    z_q = _make_gather_sc()(w_n_pad, idx_flat)
    z_q_out = z_q[:, :_D].reshape(z.shape)
    loss = (1.0 + beta) * dsum[0] / (_N * _D)
    encoding_indices = idx_flat.reshape(z.shape[:-1])
    return (z_q_out, loss, encoding_indices)


# speedup vs baseline: 1.0821x; 1.0660x over previous
import functools

import jax
import jax.numpy as jnp
from jax import lax
from jax.experimental import pallas as pl
from jax.experimental.pallas import tpu as pltpu
from jax.experimental.pallas import tpu_sc as plsc

_N = 8192
_V = 8192
_D = 64
_BM = 1024
_BN = 1024

_SC_NC = 2
_SC_NS = 16
_SC_NW = _SC_NC * _SC_NS
_B_PER_W = _N // _SC_NW
_D_PAD = 128


def _row_norm(x):
    n = jnp.sqrt(jnp.sum(x * x, axis=1, keepdims=True))
    return x / jnp.maximum(n, 1e-12)


def _argmin_body(zb_ref, w_ref, idx_ref, dsum_ref, wpad_ref, s_ref, wn_ref):
    @pl.when(pl.program_id(0) == 0)
    def _():
        wn = _row_norm(w_ref[...])
        wn_ref[...] = wn
        wpad_ref[:, :_D] = wn
        wpad_ref[:, _D:] = jnp.zeros((_V, _D_PAD - _D), jnp.float32)

    zb = _row_norm(zb_ref[...])
    m = jnp.full((_BM, 1), -jnp.inf, jnp.float32)
    for j in range(_V // _BN):
        wb = wn_ref[pl.ds(j * _BN, _BN), :]
        s = lax.dot_general(zb, wb, (((1,), (1,)), ((), ())),
                            preferred_element_type=jnp.float32)
        s_ref[:, pl.ds(j * _BN, _BN)] = s
        m = jnp.maximum(m, jnp.max(s, axis=1, keepdims=True))
    d_min = 2.0 - 2.0 * m
    colf = lax.broadcasted_iota(jnp.int32, (_BM, _BN), 1).astype(jnp.float32)
    run = jnp.full((_BM, 1), 3.0e4, jnp.float32)
    for j in range(_V // _BN):
        d = 2.0 - 2.0 * s_ref[:, pl.ds(j * _BN, _BN)]
        cand = jnp.where(d == d_min, colf, 2.0e4)
        cmin = jnp.min(cand, axis=1, keepdims=True) + float(j * _BN)
        run = jnp.minimum(run, cmin)
    idx_ref[...] = run.astype(jnp.int32)

    @pl.when(pl.program_id(0) == 0)
    def _():
        dsum_ref[0] = 0.0

    dsum_ref[0] += jnp.sum(d_min)


def _distance_argmin(z_n, w_n):
    return pl.pallas_call(
        _argmin_body,
        grid=(_N // _BM,),
        in_specs=[
            pl.BlockSpec((_BM, _D), lambda i: (i, 0)),
            pl.BlockSpec((_V, _D), lambda i: (0, 0)),
        ],
        out_specs=[
            pl.BlockSpec((_BM, 1), lambda i: (i, 0)),
            pl.BlockSpec(memory_space=pltpu.SMEM),
            pl.BlockSpec((_V, _D_PAD), lambda i: (0, 0)),
        ],
        out_shape=[
            jax.ShapeDtypeStruct((_N, 1), jnp.int32),
            jax.ShapeDtypeStruct((1,), jnp.float32),
            jax.ShapeDtypeStruct((_V, _D_PAD), jnp.float32),
        ],
        scratch_shapes=[
            pltpu.VMEM((_BM, _V), jnp.float32),
            pltpu.VMEM((_V, _D), jnp.float32),
        ],
    )(z_n, w_n)


@functools.lru_cache(maxsize=1)
def _make_gather_sc():
    @functools.partial(
        pl.kernel,
        mesh=plsc.VectorSubcoreMesh(core_axis_name="c", subcore_axis_name="s"),
        out_type=jax.ShapeDtypeStruct((_N, _D_PAD), jnp.float32),
        scratch_types=[
            pltpu.VMEM((_B_PER_W,), jnp.int32),
            pltpu.VMEM((_B_PER_W, _D_PAD), jnp.float32),
            pltpu.SemaphoreType.DMA,
        ],
    )
    def _gather_sc(table_hbm, idx_hbm, out_hbm, idx_v, rows_v, sem):
        wid = lax.axis_index("s") * _SC_NC + lax.axis_index("c")
        base = wid * _B_PER_W
        pltpu.sync_copy(idx_hbm.at[pl.ds(base, _B_PER_W)], idx_v)
        pltpu.async_copy(table_hbm.at[idx_v], rows_v, sem).wait()
        pltpu.sync_copy(rows_v, out_hbm.at[pl.ds(base, _B_PER_W)])

    return _gather_sc


def kernel(z, W):
    beta = 0.25
    idx, dsum, w_pad = _distance_argmin(z.reshape(-1, _D), W)
    idx_flat = idx.reshape(_N)
    z_q = _make_gather_sc()(w_pad, idx_flat)
    z_q_out = z_q[:, :_D].reshape(z.shape)
    loss = (1.0 + beta) * dsum[0] / (_N * _D)
    encoding_indices = idx_flat.reshape(z.shape[:-1])
    return (z_q_out, loss, encoding_indices)


# speedup vs baseline: 1.2258x; 1.1328x over previous
import functools

import jax
import jax.numpy as jnp
from jax import lax
from jax.experimental import pallas as pl
from jax.experimental.pallas import tpu as pltpu
from jax.experimental.pallas import tpu_sc as plsc

_N = 8192
_V = 8192
_D = 64
_BM = 1024
_BN = 1024

_SC_NC = 2
_SC_NS = 16
_SC_NW = _SC_NC * _SC_NS
_B_PER_W = _N // _SC_NW
_D_PAD = 128


def _row_norm(x):
    n = jnp.sqrt(jnp.sum(x * x, axis=1, keepdims=True))
    return x / jnp.maximum(n, 1e-12)


def _argmin_body(zb_ref, w_ref, idx_ref, dsum_ref, wpad_ref, s_ref, wn_ref):
    @pl.when(pl.program_id(0) == 0)
    def _():
        wn = _row_norm(w_ref[...])
        wn_ref[...] = wn
        wpad_ref[:, :_D] = wn
        wpad_ref[:, _D:] = jnp.zeros((_V, _D_PAD - _D), jnp.float32)

    zb = _row_norm(zb_ref[...])
    m = jnp.full((_BM, 1), -jnp.inf, jnp.float32)
    for j in range(_V // _BN):
        wb = wn_ref[pl.ds(j * _BN, _BN), :]
        s = lax.dot_general(zb, wb, (((1,), (1,)), ((), ())),
                            preferred_element_type=jnp.float32)
        s_ref[:, pl.ds(j * _BN, _BN)] = s
        m = jnp.maximum(m, jnp.max(s, axis=1, keepdims=True))
    d_min = 2.0 - 2.0 * m
    s0 = (2.0 - d_min) * 0.5
    b0 = lax.bitcast_convert_type(s0, jnp.int32)
    sgn = b0 < 0
    s_lo = m
    for k in range(-4, 5):
        c = lax.bitcast_convert_type(b0 + jnp.where(sgn, -k, k), jnp.float32)
        ok = (2.0 - 2.0 * c) == d_min
        s_lo = jnp.where(ok & (c < s_lo), c, s_lo)
    colf = lax.broadcasted_iota(jnp.int32, (_BM, _BN), 1).astype(jnp.float32)
    run = jnp.full((_BM, 1), 3.0e4, jnp.float32)
    for j in range(_V // _BN):
        cand = jnp.where(s_ref[:, pl.ds(j * _BN, _BN)] >= s_lo, colf, 2.0e4)
        cmin = jnp.min(cand, axis=1, keepdims=True) + float(j * _BN)
        run = jnp.minimum(run, cmin)
    idx_ref[...] = jnp.squeeze(run.astype(jnp.int32), 1)

    @pl.when(pl.program_id(0) == 0)
    def _():
        dsum_ref[0] = 0.0

    dsum_ref[0] += jnp.sum(d_min)


def _distance_argmin(z_n, w_n):
    return pl.pallas_call(
        _argmin_body,
        grid=(_N // _BM,),
        in_specs=[
            pl.BlockSpec((_BM, _D), lambda i: (i, 0)),
            pl.BlockSpec((_V, _D), lambda i: (0, 0)),
        ],
        out_specs=[
            pl.BlockSpec((_BM,), lambda i: (i,)),
            pl.BlockSpec(memory_space=pltpu.SMEM),
            pl.BlockSpec((_V, _D_PAD), lambda i: (0, 0)),
        ],
        out_shape=[
            jax.ShapeDtypeStruct((_N,), jnp.int32),
            jax.ShapeDtypeStruct((1,), jnp.float32),
            jax.ShapeDtypeStruct((_V, _D_PAD), jnp.float32),
        ],
        scratch_shapes=[
            pltpu.VMEM((_BM, _V), jnp.float32),
            pltpu.VMEM((_V, _D), jnp.float32),
        ],
    )(z_n, w_n)


@functools.lru_cache(maxsize=1)
def _make_gather_sc():
    @functools.partial(
        pl.kernel,
        mesh=plsc.VectorSubcoreMesh(core_axis_name="c", subcore_axis_name="s"),
        out_type=jax.ShapeDtypeStruct((_N, _D_PAD), jnp.float32),
        scratch_types=[
            pltpu.VMEM((_B_PER_W,), jnp.int32),
            pltpu.VMEM((_B_PER_W, _D_PAD), jnp.float32),
            pltpu.SemaphoreType.DMA,
        ],
    )
    def _gather_sc(table_hbm, idx_hbm, out_hbm, idx_v, rows_v, sem):
        wid = lax.axis_index("s") * _SC_NC + lax.axis_index("c")
        base = wid * _B_PER_W
        pltpu.sync_copy(idx_hbm.at[pl.ds(base, _B_PER_W)], idx_v)
        pltpu.async_copy(table_hbm.at[idx_v], rows_v, sem).wait()
        pltpu.sync_copy(rows_v, out_hbm.at[pl.ds(base, _B_PER_W)])

    return _gather_sc


def kernel(z, W):
    beta = 0.25
    idx, dsum, w_pad = _distance_argmin(z.reshape(-1, _D), W)
    z_q = _make_gather_sc()(w_pad, idx)
    z_q_out = z_q[:, :_D].reshape(z.shape)
    loss = (1.0 + beta) * dsum[0] / (_N * _D)
    encoding_indices = idx.reshape(z.shape[:-1])
    return (z_q_out, loss, encoding_indices)


# speedup vs baseline: 1.2279x; 1.0018x over previous
import functools

import jax
import jax.numpy as jnp
from jax import lax
from jax.experimental import pallas as pl
from jax.experimental.pallas import tpu as pltpu
from jax.experimental.pallas import tpu_sc as plsc

_N = 8192
_V = 8192
_D = 64
_BM = 512
_BN = 1024
_G = _N // _BM

_SC_NC = 2
_SC_NS = 16
_SC_NW = _SC_NC * _SC_NS
_B_PER_W = _N // _SC_NW
_D_PAD = 128


def _row_norm(x):
    n = jnp.sqrt(jnp.sum(x * x, axis=1, keepdims=True))
    return x / jnp.maximum(n, 1e-12)


def _argmin_body(zb_ref, w_ref, idx_ref, dsum_ref, wpad_ref,
                 s2_ref, m2_ref, wn_ref):
    i = pl.program_id(0)
    par = lax.rem(i, 2)

    @pl.when(i == 0)
    def _():
        wn = _row_norm(w_ref[...])
        wn_ref[...] = wn
        wpad_ref[:, :_D] = wn
        wpad_ref[:, _D:] = jnp.zeros((_V, _D_PAD - _D), jnp.float32)
        dsum_ref[0] = 0.0

    @pl.when(i < _G)
    def _():
        zb = _row_norm(zb_ref[...])
        m = jnp.full((_BM, 1), -jnp.inf, jnp.float32)
        for j in range(_V // _BN):
            wb = wn_ref[pl.ds(j * _BN, _BN), :]
            s = lax.dot_general(zb, wb, (((1,), (1,)), ((), ())),
                                preferred_element_type=jnp.float32)
            s2_ref[pl.ds(par, 1), :, pl.ds(j * _BN, _BN)] = s[None]
            m = jnp.maximum(m, jnp.max(s, axis=1, keepdims=True))
        m2_ref[pl.ds(par, 1)] = m[None]

    @pl.when(i > 0)
    def _():
        q = 1 - par
        m = jnp.squeeze(m2_ref[pl.ds(q, 1)], 0)
        d_min = 2.0 - 2.0 * m
        s0 = (2.0 - d_min) * 0.5
        b0 = lax.bitcast_convert_type(s0, jnp.int32)
        sgn = b0 < 0
        s_lo = m
        for k in range(-4, 5):
            c = lax.bitcast_convert_type(b0 + jnp.where(sgn, -k, k),
                                         jnp.float32)
            ok = (2.0 - 2.0 * c) == d_min
            s_lo = jnp.where(ok & (c < s_lo), c, s_lo)
        colf = lax.broadcasted_iota(jnp.int32, (_BM, _BN), 1).astype(jnp.float32)
        run = jnp.full((_BM, 1), 3.0e4, jnp.float32)
        for j in range(_V // _BN):
            t = jnp.squeeze(s2_ref[pl.ds(q, 1), :, pl.ds(j * _BN, _BN)], 0)
            cand = jnp.where(t >= s_lo, colf, 2.0e4)
            cmin = jnp.min(cand, axis=1, keepdims=True) + float(j * _BN)
            run = jnp.minimum(run, cmin)
        idx_ref[...] = jnp.squeeze(run.astype(jnp.int32), 1)
        dsum_ref[0] += jnp.sum(d_min)


def _distance_argmin(z_n, w_n):
    return pl.pallas_call(
        _argmin_body,
        grid=(_G + 1,),
        in_specs=[
            pl.BlockSpec((_BM, _D), lambda i: (jnp.minimum(i, _G - 1), 0)),
            pl.BlockSpec((_V, _D), lambda i: (0, 0)),
        ],
        out_specs=[
            pl.BlockSpec((_BM,), lambda i: (jnp.maximum(i - 1, 0),)),
            pl.BlockSpec(memory_space=pltpu.SMEM),
            pl.BlockSpec((_V, _D_PAD), lambda i: (0, 0)),
        ],
        out_shape=[
            jax.ShapeDtypeStruct((_N,), jnp.int32),
            jax.ShapeDtypeStruct((1,), jnp.float32),
            jax.ShapeDtypeStruct((_V, _D_PAD), jnp.float32),
        ],
        scratch_shapes=[
            pltpu.VMEM((2, _BM, _V), jnp.float32),
            pltpu.VMEM((2, _BM, 1), jnp.float32),
            pltpu.VMEM((_V, _D), jnp.float32),
        ],
    )(z_n, w_n)


@functools.lru_cache(maxsize=1)
def _make_gather_sc():
    @functools.partial(
        pl.kernel,
        mesh=plsc.VectorSubcoreMesh(core_axis_name="c", subcore_axis_name="s"),
        out_type=jax.ShapeDtypeStruct((_N, _D_PAD), jnp.float32),
        scratch_types=[
            pltpu.VMEM((_B_PER_W,), jnp.int32),
            pltpu.VMEM((_B_PER_W, _D_PAD), jnp.float32),
            pltpu.SemaphoreType.DMA,
        ],
    )
    def _gather_sc(table_hbm, idx_hbm, out_hbm, idx_v, rows_v, sem):
        wid = lax.axis_index("s") * _SC_NC + lax.axis_index("c")
        base = wid * _B_PER_W
        pltpu.sync_copy(idx_hbm.at[pl.ds(base, _B_PER_W)], idx_v)
        pltpu.async_copy(table_hbm.at[idx_v], rows_v, sem).wait()
        pltpu.sync_copy(rows_v, out_hbm.at[pl.ds(base, _B_PER_W)])

    return _gather_sc


def kernel(z, W):
    beta = 0.25
    idx, dsum, w_pad = _distance_argmin(z.reshape(-1, _D), W)
    z_q = _make_gather_sc()(w_pad, idx)
    z_q_out = z_q[:, :_D].reshape(z.shape)
    loss = (1.0 + beta) * dsum[0] / (_N * _D)
    encoding_indices = idx.reshape(z.shape[:-1])
    return (z_q_out, loss, encoding_indices)
